# baseline (speedup 1.0000x reference)
import jax
import jax.numpy as jnp
from jax import lax
from jax.experimental import pallas as pl
from jax.experimental.pallas import tpu as pltpu
from jax.experimental.pallas import tpu_sc as plsc

H = 512
W = 512
HW = H * W
Q = HW // 4
NBLK = 32
BLK = Q // NBLK
TH = jnp.float32(1e-5)


def _tree_sum(v, lanes):
    for s in (1, 2, 4, 8):
        v = v + v[(lanes + s) % 16]
    return v


def _tree_min(v, lanes):
    for s in (1, 2, 4, 8):
        v = jnp.minimum(v, v[(lanes + s) % 16])
    return v


def _prefix(v, lanes):
    for s in (1, 2, 4, 8):
        v = v + jnp.where(lanes >= s, v[(lanes - s) % 16], 0.0)
    return v


def _sc_mags_kernel(loc_ref, out_ref, buf_ref, stats_ref, gstats_ref,
                    outv_ref, shared_ref):
    cid = lax.axis_index("c")
    sid = lax.axis_index("s")
    grp = sid // 4
    quarter = sid % 4
    sample = cid * 4 + grp
    lanes = lax.iota(jnp.int32, 16)
    lanes_f = lanes.astype(jnp.float32)

    base = sample * HW + quarter * Q
    pltpu.sync_copy(loc_ref.at[pl.ds(base, Q)], buf_ref)
    running = jnp.zeros((16,), jnp.float32)
    v0 = jnp.zeros((16,), jnp.float32)
    v1 = jnp.zeros((16,), jnp.float32)
    for blk in range(NBLK):
        def body(it, acc, blk=blk):
            off = blk * BLK + it * 256
            for u in range(16):
                x = buf_ref[pl.ds(off + u * 16, 16)]
                acc = acc + jnp.where(x >= TH, 1.0, 0.0)
            return acc
        acc = lax.fori_loop(0, BLK // 256, body, jnp.zeros((16,), jnp.float32))
        running = running + _tree_sum(acc, lanes)
        if blk < 16:
            v0 = jnp.where(lanes == blk, running, v0)
        else:
            v1 = jnp.where(lanes == blk - 16, running, v1)
    stats_ref[pl.ds(0, 16)] = v0
    stats_ref[pl.ds(16, 16)] = v1
    pltpu.sync_copy(stats_ref, shared_ref.at[pl.ds(sid * 32, 32)])
    plsc.subcore_barrier()

    @pl.when(quarter == 0)
    def _():
        pltpu.sync_copy(shared_ref.at[pl.ds(grp * 4 * 32, 4 * 32)], gstats_ref)
        qt = [gstats_ref[pl.ds(q * 32 + 16, 16)][15] for q in range(4)]
        nf = qt[0] + qt[1] + qt[2] + qt[3]
        n = nf.astype(jnp.int32)
        target = (n // 2 + 1).astype(jnp.float32)
        e1 = qt[0]
        e2 = e1 + qt[1]
        e3 = e2 + qt[2]
        qstar = jnp.where(target <= e1, 0,
                          jnp.where(target <= e2, 1,
                                    jnp.where(target <= e3, 2, 3)))
        eq = jnp.where(qstar == 0, 0.0,
                       jnp.where(qstar == 1, e1,
                                 jnp.where(qstar == 2, e2, e3)))
        r1 = target - eq

        va = gstats_ref[pl.ds(qstar * 32, 16)]
        vb = gstats_ref[pl.ds(qstar * 32 + 16, 16)]
        minA = _tree_min(jnp.where(va >= r1, lanes_f, 100.0), lanes)[0]
        minB = _tree_min(jnp.where(vb >= r1, lanes_f, 100.0), lanes)[0]
        bstar = jnp.where(minA < 100.0, minA, 16.0 + minB).astype(jnp.int32)
        bm1 = bstar - 1
        pick = jnp.where(bm1 < 16, bm1, bm1 - 16)
        vsel = jnp.where(bm1 < 16, va, vb)
        prev_v = _tree_sum(jnp.where(lanes == pick, vsel, 0.0), lanes)
        prev = jnp.where(bstar == 0, 0.0, prev_v[0])
        r2 = r1 - prev

        off = sample * HW + qstar * Q + bstar * BLK
        off = pl.multiple_of(off, 8)
        pltpu.sync_copy(loc_ref.at[pl.ds(off, BLK)], buf_ref.at[pl.ds(0, BLK)])

        def sbody(it, carry):
            cnt, fidx, found = carry
            x = buf_ref[pl.ds(it * 16, 16)]
            msk = x >= TH
            ci = jnp.where(msk, 1.0, 0.0)
            incl = _prefix(ci, lanes)
            tot = incl[15]
            hitm = jnp.logical_and(msk, incl == (r2 - cnt))
            lane = _tree_min(jnp.where(hitm, lanes_f, 100.0), lanes)[0]
            fh = jnp.logical_and(jnp.logical_not(found), lane < 100.0)
            fidx = jnp.where(fh, it * 16 + lane.astype(jnp.int32), fidx)
            return (cnt + tot, fidx, jnp.logical_or(found, fh))
        _, lidx, _ = lax.fori_loop(0, BLK // 16, sbody,
                                   (jnp.float32(0), jnp.int32(0),
                                    jnp.bool_(False)))
        flat = qstar * Q + bstar * BLK + lidx
        flat = jnp.where(n == 0, 0, flat)
        row = flat // W
        col = flat - row * W
        outv_ref[...] = jnp.where(lanes == 0, row - H // 2,
                                  jnp.where(lanes == 1, col - W // 2, 0))
        pltpu.sync_copy(outv_ref, out_ref.at[pl.ds(sample * 16, 16)])


def _sc_mags(loc_flat, batch):
    mesh = plsc.VectorSubcoreMesh(core_axis_name="c", subcore_axis_name="s",
                                  num_cores=2, num_subcores=16)
    return pl.kernel(
        _sc_mags_kernel,
        out_type=jax.ShapeDtypeStruct((batch * 16,), jnp.int32),
        mesh=mesh,
        scratch_types=[
            pltpu.VMEM((Q,), jnp.float32),
            pltpu.VMEM((32,), jnp.float32),
            pltpu.VMEM((4 * 32,), jnp.float32),
            pltpu.VMEM((16,), jnp.int32),
            pltpu.VMEM_SHARED((16 * 32,), jnp.float32),
        ],
    )(loc_flat)


def _roll2d(src, typ, txp):
    return pltpu.roll(pltpu.roll(src, txp, axis=1), typ, axis=0)


def _tc_kernel(mags_ref, bkg_ref, bkgm_ref, defect_ref, dmask_ref,
               img_ref, outm_ref):
    b = pl.program_id(0)
    ty = mags_ref[b * 16]
    tx = mags_ref[b * 16 + 1]
    i = lax.broadcasted_iota(jnp.int32, (H, W), 0)
    j = lax.broadcasted_iota(jnp.int32, (H, W), 1)
    txp = jnp.where(tx < 0, tx + W, tx)
    typ = jnp.where(ty < 0, ty + H, ty)
    valid = (j >= tx) & (j < W + tx) & (i >= ty) & (i < H + ty)
    dm = jnp.where(valid, _roll2d(dmask_ref[0, 0], typ, txp), 0.0)
    for c in range(3):
        d_s = _roll2d(defect_ref[0, c], typ, txp)
        img_ref[0, c] = bkg_ref[0, c] * (1.0 - dm) + d_s * dm
    outm_ref[0, 0] = jnp.clip(dm + bkgm_ref[0, 0], 0.0, 1.0)


@jax.jit
def kernel(bkg, bkg_mask, defect, defect_mask, defect_location_masks):
    B = bkg.shape[0]
    mags = _sc_mags(defect_location_masks.reshape(B * HW), B)

    def ch_map(b):
        return (b, 0, 0, 0)

    img_aug, out_mask = pl.pallas_call(
        _tc_kernel,
        grid=(B,),
        in_specs=[
            pl.BlockSpec(memory_space=pltpu.SMEM),
            pl.BlockSpec((1, 3, H, W), ch_map),
            pl.BlockSpec((1, 1, H, W), ch_map),
            pl.BlockSpec((1, 3, H, W), ch_map),
            pl.BlockSpec((1, 1, H, W), ch_map),
        ],
        out_specs=[
            pl.BlockSpec((1, 3, H, W), ch_map),
            pl.BlockSpec((1, 1, H, W), ch_map),
        ],
        out_shape=[
            jax.ShapeDtypeStruct((B, 3, H, W), jnp.float32),
            jax.ShapeDtypeStruct((B, 1, H, W), jnp.float32),
        ],
    )(mags, bkg, bkg_mask, defect, defect_mask)
    return img_aug, out_mask


# speedup vs baseline: 1.0920x; 1.0920x over previous
import jax
import jax.numpy as jnp
from jax import lax
from jax.experimental import pallas as pl
from jax.experimental.pallas import tpu as pltpu
from jax.experimental.pallas import tpu_sc as plsc

H = 512
W = 512
HW = H * W
Q = HW // 4
NBLK = 32
BLK = Q // NBLK
TH = 1e-5


def _tree_sum(v, lanes):
    for s in (1, 2, 4, 8):
        v = v + v[(lanes + s) % 16]
    return v


def _tree_min(v, lanes):
    for s in (1, 2, 4, 8):
        v = jnp.minimum(v, v[(lanes + s) % 16])
    return v


def _prefix(v, lanes):
    for s in (1, 2, 4, 8):
        v = v + jnp.where(lanes >= s, v[(lanes - s) % 16], 0.0)
    return v


def _sc_mags_kernel(loc_ref, out_ref, buf_ref, stats_ref, gstats_ref,
                    outv_ref, shared_ref):
    cid = lax.axis_index("c")
    sid = lax.axis_index("s")
    grp = sid // 4
    quarter = sid % 4
    sample = cid * 4 + grp
    lanes = lax.iota(jnp.int32, 16)
    lanes_f = lanes.astype(jnp.float32)

    pltpu.sync_copy(loc_ref.at[sample, 0, pl.ds(quarter * 128, 128), :],
                    buf_ref)
    running = jnp.zeros((16,), jnp.float32)
    v0 = jnp.zeros((16,), jnp.float32)
    v1 = jnp.zeros((16,), jnp.float32)
    for blk in range(NBLK):
        acc = jnp.zeros((16,), jnp.float32)
        for dr in range(4):
            def body(it, acc, blk=blk, dr=dr):
                off = it * 128
                for u in range(8):
                    x = buf_ref[blk * 4 + dr, pl.ds(off + u * 16, 16)]
                    acc = acc + jnp.where(x >= TH, 1.0, 0.0)
                return acc
            acc = lax.fori_loop(0, 4, body, acc)
        running = running + _tree_sum(acc, lanes)
        if blk < 16:
            v0 = jnp.where(lanes == blk, running, v0)
        else:
            v1 = jnp.where(lanes == blk - 16, running, v1)
    stats_ref[pl.ds(0, 16)] = v0
    stats_ref[pl.ds(16, 16)] = v1
    pltpu.sync_copy(stats_ref, shared_ref.at[pl.ds(sid * 32, 32)])
    plsc.subcore_barrier()

    @pl.when(quarter == 0)
    def _():
        pltpu.sync_copy(shared_ref.at[pl.ds(grp * 4 * 32, 4 * 32)], gstats_ref)
        qt = [gstats_ref[pl.ds(q * 32 + 16, 16)][15] for q in range(4)]
        nf = qt[0] + qt[1] + qt[2] + qt[3]
        n = nf.astype(jnp.int32)
        target = (n // 2 + 1).astype(jnp.float32)
        e1 = qt[0]
        e2 = e1 + qt[1]
        e3 = e2 + qt[2]
        qstar = jnp.where(target <= e1, 0,
                          jnp.where(target <= e2, 1,
                                    jnp.where(target <= e3, 2, 3)))
        eq = jnp.where(qstar == 0, 0.0,
                       jnp.where(qstar == 1, e1,
                                 jnp.where(qstar == 2, e2, e3)))
        r1 = target - eq

        va = gstats_ref[pl.ds(qstar * 32, 16)]
        vb = gstats_ref[pl.ds(qstar * 32 + 16, 16)]
        minA = _tree_min(jnp.where(va >= r1, lanes_f, 100.0), lanes)[0]
        minB = _tree_min(jnp.where(vb >= r1, lanes_f, 100.0), lanes)[0]
        bstar = jnp.where(minA < 100.0, minA, 16.0 + minB).astype(jnp.int32)
        bm1 = bstar - 1
        pick = jnp.where(bm1 < 16, bm1, bm1 - 16)
        vsel = jnp.where(bm1 < 16, va, vb)
        prev_v = _tree_sum(jnp.where(lanes == pick, vsel, 0.0), lanes)
        prev = jnp.where(bstar == 0, 0.0, prev_v[0])
        r2 = r1 - prev

        r0 = qstar * 128 + bstar * 4
        pltpu.sync_copy(loc_ref.at[sample, 0, pl.ds(r0, 4), :],
                        buf_ref.at[pl.ds(0, 4), :])

        def sbody(it, carry):
            cnt, fidx, found = carry
            x = buf_ref[it // 32, pl.ds((it % 32) * 16, 16)]
            msk = x >= TH
            ci = jnp.where(msk, 1.0, 0.0)
            incl = _prefix(ci, lanes)
            tot = incl[15]
            hitm = jnp.logical_and(msk, incl == (r2 - cnt))
            lane = _tree_min(jnp.where(hitm, lanes_f, 100.0), lanes)[0]
            fh = jnp.logical_and(jnp.logical_not(found), lane < 100.0)
            fidx = jnp.where(fh, it * 16 + lane.astype(jnp.int32), fidx)
            return (cnt + tot, fidx, jnp.logical_or(found, fh))
        _, lidx, _ = lax.fori_loop(0, BLK // 16, sbody,
                                   (jnp.float32(0), jnp.int32(0),
                                    jnp.bool_(False)))
        flat = qstar * Q + bstar * BLK + lidx
        flat = jnp.where(n == 0, 0, flat)
        row = flat // W
        col = flat - row * W
        outv_ref[...] = jnp.where(lanes == 0, row - H // 2,
                                  jnp.where(lanes == 1, col - W // 2, 0))
        pltpu.sync_copy(outv_ref, out_ref.at[pl.ds(sample * 16, 16)])


def _sc_mags(loc, batch):
    mesh = plsc.VectorSubcoreMesh(core_axis_name="c", subcore_axis_name="s",
                                  num_cores=2, num_subcores=16)
    return pl.kernel(
        _sc_mags_kernel,
        out_type=jax.ShapeDtypeStruct((batch * 16,), jnp.int32),
        mesh=mesh,
        scratch_types=[
            pltpu.VMEM((128, 512), jnp.float32),
            pltpu.VMEM((32,), jnp.float32),
            pltpu.VMEM((4 * 32,), jnp.float32),
            pltpu.VMEM((16,), jnp.int32),
            pltpu.VMEM_SHARED((16 * 32,), jnp.float32),
        ],
    )(loc)


def _roll2d(src, typ, txp):
    return pltpu.roll(pltpu.roll(src, txp, axis=1), typ, axis=0)


def _tc_kernel(mags_ref, bkg_ref, bkgm_ref, defect_ref, dmask_ref,
               img_ref, outm_ref):
    b = pl.program_id(0)
    ty = mags_ref[b * 16]
    tx = mags_ref[b * 16 + 1]
    i = lax.broadcasted_iota(jnp.int32, (H, W), 0)
    j = lax.broadcasted_iota(jnp.int32, (H, W), 1)
    txp = jnp.where(tx < 0, tx + W, tx)
    typ = jnp.where(ty < 0, ty + H, ty)
    valid = (j >= tx) & (j < W + tx) & (i >= ty) & (i < H + ty)
    dm = jnp.where(valid, _roll2d(dmask_ref[0, 0], typ, txp), 0.0)
    for c in range(3):
        d_s = _roll2d(defect_ref[0, c], typ, txp)
        img_ref[0, c] = bkg_ref[0, c] * (1.0 - dm) + d_s * dm
    outm_ref[0, 0] = jnp.clip(dm + bkgm_ref[0, 0], 0.0, 1.0)


@jax.jit
def kernel(bkg, bkg_mask, defect, defect_mask, defect_location_masks):
    B = bkg.shape[0]
    mags = _sc_mags(defect_location_masks, B)

    def ch_map(b):
        return (b, 0, 0, 0)

    img_aug, out_mask = pl.pallas_call(
        _tc_kernel,
        grid=(B,),
        in_specs=[
            pl.BlockSpec(memory_space=pltpu.SMEM),
            pl.BlockSpec((1, 3, H, W), ch_map),
            pl.BlockSpec((1, 1, H, W), ch_map),
            pl.BlockSpec((1, 3, H, W), ch_map),
            pl.BlockSpec((1, 1, H, W), ch_map),
        ],
        out_specs=[
            pl.BlockSpec((1, 3, H, W), ch_map),
            pl.BlockSpec((1, 1, H, W), ch_map),
        ],
        out_shape=[
            jax.ShapeDtypeStruct((B, 3, H, W), jnp.float32),
            jax.ShapeDtypeStruct((B, 1, H, W), jnp.float32),
        ],
    )(mags, bkg, bkg_mask, defect, defect_mask)
    return img_aug, out_mask


# speedup vs baseline: 1.1248x; 1.0300x over previous
import jax
import jax.numpy as jnp
from jax import lax
from jax.experimental import pallas as pl
from jax.experimental.pallas import tpu as pltpu
from jax.experimental.pallas import tpu_sc as plsc

H = 512
W = 512
HW = H * W
Q = HW // 4
NBLK = 32
BLK = Q // NBLK
TH = 1e-5


def _tree_sum(v, lanes):
    for s in (1, 2, 4, 8):
        v = v + v[(lanes + s) % 16]
    return v


def _tree_min(v, lanes):
    for s in (1, 2, 4, 8):
        v = jnp.minimum(v, v[(lanes + s) % 16])
    return v


def _prefix(v, lanes):
    for s in (1, 2, 4, 8):
        v = v + jnp.where(lanes >= s, v[(lanes - s) % 16], 0.0)
    return v


def _sc_mags_kernel(loc_ref, out_ref, buf_ref, stats_ref, gstats_ref,
                    outv_ref, shared_ref, sems):
    cid = lax.axis_index("c")
    sid = lax.axis_index("s")
    grp = sid // 4
    quarter = sid % 4
    sample = cid * 4 + grp
    lanes = lax.iota(jnp.int32, 16)
    lanes_f = lanes.astype(jnp.float32)

    copies = [
        pltpu.make_async_copy(
            loc_ref.at[sample, 0, pl.ds(quarter * 128 + p * 32, 32), :],
            buf_ref.at[pl.ds(p * 32, 32), :], sems.at[p])
        for p in range(4)
    ]
    for cp in copies:
        cp.start()
    running = jnp.zeros((16,), jnp.float32)
    v0 = jnp.zeros((16,), jnp.float32)
    v1 = jnp.zeros((16,), jnp.float32)
    for blk in range(NBLK):
        if blk % 8 == 0:
            copies[blk // 8].wait()
        acc = jnp.zeros((16,), jnp.float32)
        for dr in range(4):
            def body(it, accs, blk=blk, dr=dr):
                off = it * 256
                accs = list(accs)
                for u in range(16):
                    x = buf_ref[blk * 4 + dr, pl.ds(off + u * 16, 16)]
                    accs[u % 2] = accs[u % 2] + jnp.where(x >= TH, 1.0, 0.0)
                return tuple(accs)
            a0, a1 = lax.fori_loop(0, 2, body,
                                   (acc, jnp.zeros((16,), jnp.float32)))
            acc = a0 + a1
        running = running + _tree_sum(acc, lanes)
        if blk < 16:
            v0 = jnp.where(lanes == blk, running, v0)
        else:
            v1 = jnp.where(lanes == blk - 16, running, v1)
    stats_ref[pl.ds(0, 16)] = v0
    stats_ref[pl.ds(16, 16)] = v1
    pltpu.sync_copy(stats_ref, shared_ref.at[pl.ds(sid * 32, 32)])
    plsc.subcore_barrier()

    @pl.when(quarter == 0)
    def _():
        pltpu.sync_copy(shared_ref.at[pl.ds(grp * 4 * 32, 4 * 32)], gstats_ref)
        qt = [gstats_ref[pl.ds(q * 32 + 16, 16)][15] for q in range(4)]
        nf = qt[0] + qt[1] + qt[2] + qt[3]
        n = nf.astype(jnp.int32)
        target = (n // 2 + 1).astype(jnp.float32)
        e1 = qt[0]
        e2 = e1 + qt[1]
        e3 = e2 + qt[2]
        qstar = jnp.where(target <= e1, 0,
                          jnp.where(target <= e2, 1,
                                    jnp.where(target <= e3, 2, 3)))
        eq = jnp.where(qstar == 0, 0.0,
                       jnp.where(qstar == 1, e1,
                                 jnp.where(qstar == 2, e2, e3)))
        r1 = target - eq

        va = gstats_ref[pl.ds(qstar * 32, 16)]
        vb = gstats_ref[pl.ds(qstar * 32 + 16, 16)]
        minA = _tree_min(jnp.where(va >= r1, lanes_f, 100.0), lanes)[0]
        minB = _tree_min(jnp.where(vb >= r1, lanes_f, 100.0), lanes)[0]
        bstar = jnp.where(minA < 100.0, minA, 16.0 + minB).astype(jnp.int32)
        bm1 = bstar - 1
        pick = jnp.where(bm1 < 16, bm1, bm1 - 16)
        vsel = jnp.where(bm1 < 16, va, vb)
        prev_v = _tree_sum(jnp.where(lanes == pick, vsel, 0.0), lanes)
        prev = jnp.where(bstar == 0, 0.0, prev_v[0])
        r2 = r1 - prev

        r0 = qstar * 128 + bstar * 4
        pltpu.sync_copy(loc_ref.at[sample, 0, pl.ds(r0, 4), :],
                        buf_ref.at[pl.ds(0, 4), :])

        def sbody(it, carry):
            cnt, fidx, found = carry
            x = buf_ref[it // 32, pl.ds((it % 32) * 16, 16)]
            msk = x >= TH
            ci = jnp.where(msk, 1.0, 0.0)
            incl = _prefix(ci, lanes)
            tot = incl[15]
            hitm = jnp.logical_and(msk, incl == (r2 - cnt))
            lane = _tree_min(jnp.where(hitm, lanes_f, 100.0), lanes)[0]
            fh = jnp.logical_and(jnp.logical_not(found), lane < 100.0)
            fidx = jnp.where(fh, it * 16 + lane.astype(jnp.int32), fidx)
            return (cnt + tot, fidx, jnp.logical_or(found, fh))
        _, lidx, _ = lax.fori_loop(0, BLK // 16, sbody,
                                   (jnp.float32(0), jnp.int32(0),
                                    jnp.bool_(False)))
        flat = qstar * Q + bstar * BLK + lidx
        flat = jnp.where(n == 0, 0, flat)
        row = flat // W
        col = flat - row * W
        outv_ref[...] = jnp.where(lanes == 0, row - H // 2,
                                  jnp.where(lanes == 1, col - W // 2, 0))
        pltpu.sync_copy(outv_ref, out_ref.at[pl.ds(sample * 16, 16)])


def _sc_mags(loc, batch):
    mesh = plsc.VectorSubcoreMesh(core_axis_name="c", subcore_axis_name="s",
                                  num_cores=2, num_subcores=16)
    return pl.kernel(
        _sc_mags_kernel,
        out_type=jax.ShapeDtypeStruct((batch * 16,), jnp.int32),
        mesh=mesh,
        scratch_types=[
            pltpu.VMEM((128, 512), jnp.float32),
            pltpu.VMEM((32,), jnp.float32),
            pltpu.VMEM((4 * 32,), jnp.float32),
            pltpu.VMEM((16,), jnp.int32),
            pltpu.VMEM_SHARED((16 * 32,), jnp.float32),
            pltpu.SemaphoreType.DMA((4,)),
        ],
    )(loc)


def _roll2d(src, typ, txp):
    return pltpu.roll(pltpu.roll(src, txp, axis=1), typ, axis=0)


def _tc_kernel(mags_ref, bkg_ref, bkgm_ref, defect_ref, dmask_ref,
               img_ref, outm_ref):
    b = pl.program_id(0)
    ty = mags_ref[b * 16]
    tx = mags_ref[b * 16 + 1]
    i = lax.broadcasted_iota(jnp.int32, (H, W), 0)
    j = lax.broadcasted_iota(jnp.int32, (H, W), 1)
    txp = jnp.where(tx < 0, tx + W, tx)
    typ = jnp.where(ty < 0, ty + H, ty)
    valid = (j >= tx) & (j < W + tx) & (i >= ty) & (i < H + ty)
    dm = jnp.where(valid, _roll2d(dmask_ref[0, 0], typ, txp), 0.0)
    for c in range(3):
        d_s = _roll2d(defect_ref[0, c], typ, txp)
        img_ref[0, c] = bkg_ref[0, c] * (1.0 - dm) + d_s * dm
    outm_ref[0, 0] = jnp.clip(dm + bkgm_ref[0, 0], 0.0, 1.0)


@jax.jit
def kernel(bkg, bkg_mask, defect, defect_mask, defect_location_masks):
    B = bkg.shape[0]
    mags = _sc_mags(defect_location_masks, B)

    def ch_map(b):
        return (b, 0, 0, 0)

    img_aug, out_mask = pl.pallas_call(
        _tc_kernel,
        grid=(B,),
        in_specs=[
            pl.BlockSpec(memory_space=pltpu.SMEM),
            pl.BlockSpec((1, 3, H, W), ch_map),
            pl.BlockSpec((1, 1, H, W), ch_map),
            pl.BlockSpec((1, 3, H, W), ch_map),
            pl.BlockSpec((1, 1, H, W), ch_map),
        ],
        out_specs=[
            pl.BlockSpec((1, 3, H, W), ch_map),
            pl.BlockSpec((1, 1, H, W), ch_map),
        ],
        out_shape=[
            jax.ShapeDtypeStruct((B, 3, H, W), jnp.float32),
            jax.ShapeDtypeStruct((B, 1, H, W), jnp.float32),
        ],
    )(mags, bkg, bkg_mask, defect, defect_mask)
    return img_aug, out_mask
